# baseline (speedup 1.0000x reference)
import functools

import jax
import jax.numpy as jnp
from jax import lax
from jax.experimental import pallas as pl
from jax.experimental.pallas import tpu as pltpu
from jax.experimental.pallas import tpu_sc as plsc

B = 16384
V = 1000001
EMB = 32
SK = 20
OUT_W = 568
NI = OUT_W // 8
NJ = B // 128
NC = 2
NS = 16
L = 16
NW = NC * NS
JPW = NJ // NW
CH = 64


def _assemble(buf, irows_v, emp_v, lat_v, lon_v, sk_v, iota, ones):
    for g in range(CH // L):
        rows = g * L + iota
        for c in range(EMB):
            v = plsc.load_gather(irows_v, [rows, jnp.full((L,), c, jnp.int32)])
            buf[4 + c // 8, c % 8, pl.ds(g * L, L)] = v
        buf[8, 0, pl.ds(g * L, L)] = emp_v[pl.ds(g * L, L)]
        buf[8, 1, pl.ds(g * L, L)] = lat_v[pl.ds(g * L, L)]
        buf[8, 2, pl.ds(g * L, L)] = lon_v[pl.ds(g * L, L)]
        for k in range(SK):
            sk = plsc.load_gather(sk_v, [rows, jnp.full((L,), k, jnp.int32)])
            c = 67 + sk
            plsc.store_scatter(buf, [lax.shift_right_logical(c, 3),
                                     lax.bitwise_and(c, 7), rows], ones)


def _clear(buf, sk_v, iota, zeros):
    for g in range(CH // L):
        rows = g * L + iota
        for k in range(SK):
            sk = plsc.load_gather(sk_v, [rows, jnp.full((L,), k, jnp.int32)])
            c = 67 + sk
            plsc.store_scatter(buf, [lax.shift_right_logical(c, 3),
                                     lax.bitwise_and(c, 7), rows], zeros)


def _sc_body(lid_hbm, ind_hbm, emp_hbm, lat_hbm, lon_hbm, sk_hbm,
             ltab_hbm, itab_hbm, out_hbm,
             lid_v, ind_v, emp_v, lat_v, lon_v, sk0_v, sk1_v,
             idx_v, irows_v, buf0, buf1,
             sem_in, sem_g, sem_o0, sem_o1):
    wid = lax.axis_index("s") * NC + lax.axis_index("c")
    iota = lax.iota(jnp.int32, L)
    ones = jnp.full((L,), 1.0, jnp.float32)
    zeros = jnp.zeros((L,), jnp.float32)

    def zero_i(i, _):
        for k in range(8):
            for g in range(CH // L):
                buf0[i, k, pl.ds(g * L, L)] = zeros
                buf1[i, k, pl.ds(g * L, L)] = zeros
        return 0

    lax.fori_loop(8, NI, zero_i, 0)

    def do_chunk(jj, l0, buf, sk_v, sem_o, first):
        base = jj * 128 + l0
        @pl.when(jnp.logical_not(first))
        def _():
            pltpu.make_async_copy(
                buf, out_hbm.at[:, jj, :, pl.ds(l0, CH)], sem_o).wait()
            _clear(buf, sk_v, iota, zeros)

        cps = [
            pltpu.async_copy(lid_hbm.at[pl.ds(base, CH)], lid_v, sem_in),
            pltpu.async_copy(ind_hbm.at[pl.ds(base, CH)], ind_v, sem_in),
            pltpu.async_copy(emp_hbm.at[pl.ds(base, CH)], emp_v, sem_in),
            pltpu.async_copy(lat_hbm.at[pl.ds(base, CH)], lat_v, sem_in),
            pltpu.async_copy(lon_hbm.at[pl.ds(base, CH)], lon_v, sem_in),
            pltpu.async_copy(sk_hbm.at[pl.ds(base, CH)], sk_v, sem_in),
        ]
        for cp in cps:
            cp.wait()
        for c in range(EMB):
            for g in range(CH // L):
                lid16 = lid_v[pl.ds(g * L, L)]
                idx_v[c, pl.ds(g * L, L)] = lid16 + jnp.int32(c * V)
        gs = [pltpu.async_copy(ltab_hbm.at[idx_v.at[c]],
                               buf.at[c // 8, c % 8], sem_g)
              for c in range(EMB)]
        g2 = pltpu.async_copy(itab_hbm.at[ind_v], irows_v, sem_g)
        for cp in gs:
            cp.wait()
        g2.wait()
        _assemble(buf, irows_v, emp_v, lat_v, lon_v, sk_v, iota, ones)
        pltpu.async_copy(buf, out_hbm.at[:, jj, :, pl.ds(l0, CH)], sem_o)

    def do_pair(i, _):
        jj = wid * JPW + i
        first = i == 0
        do_chunk(jj, 0, buf0, sk0_v, sem_o0, first)
        do_chunk(jj, CH, buf1, sk1_v, sem_o1, first)
        return 0

    lax.fori_loop(0, JPW, do_pair, 0)
    last_j = wid * JPW + JPW - 1
    pltpu.make_async_copy(
        buf0, out_hbm.at[:, last_j, :, pl.ds(0, CH)], sem_o0).wait()
    pltpu.make_async_copy(
        buf1, out_hbm.at[:, last_j, :, pl.ds(CH, CH)], sem_o1).wait()


@jax.jit
def _run(lid, ind, emp, lat, lon, sk, ltab_lin, itab):
    mesh = plsc.VectorSubcoreMesh(core_axis_name="c", subcore_axis_name="s")
    f = functools.partial(
        pl.kernel,
        mesh=mesh,
        compiler_params=pltpu.CompilerParams(use_tc_tiling_on_sc=False,
                                             needs_layout_passes=False),
        out_type=jax.ShapeDtypeStruct((NI, NJ, 8, 128), jnp.float32),
        scratch_types=[
            pltpu.VMEM((CH,), jnp.int32),
            pltpu.VMEM((CH,), jnp.int32),
            pltpu.VMEM((CH,), jnp.float32),
            pltpu.VMEM((CH,), jnp.float32),
            pltpu.VMEM((CH,), jnp.float32),
            pltpu.VMEM((CH, SK), jnp.int32),
            pltpu.VMEM((CH, SK), jnp.int32),
            pltpu.VMEM((EMB, CH), jnp.int32),
            pltpu.VMEM((CH, EMB), jnp.float32),
            pltpu.VMEM((NI, 8, CH), jnp.float32),
            pltpu.VMEM((NI, 8, CH), jnp.float32),
            pltpu.SemaphoreType.DMA,
            pltpu.SemaphoreType.DMA,
            pltpu.SemaphoreType.DMA,
            pltpu.SemaphoreType.DMA,
        ],
    )(_sc_body)
    out4 = f(lid, ind, emp, lat, lon, sk, ltab_lin, itab)
    return jnp.transpose(out4, (1, 3, 0, 2)).reshape(B, OUT_W)


def kernel(listing_id, listing_industry_type, employer_num_employees,
           listing_loc_latitude, listing_loc_longitude, listing_skills,
           listing_table, industry_table):
    tab_lin = listing_table.T.reshape(-1)
    return _run(listing_id.astype(jnp.int32),
                listing_industry_type.astype(jnp.int32),
                employer_num_employees,
                listing_loc_latitude,
                listing_loc_longitude,
                listing_skills.astype(jnp.int32),
                tab_lin, industry_table)


# speedup vs baseline: 4.6258x; 4.6258x over previous
import functools

import jax
import jax.numpy as jnp
from jax import lax
from jax.experimental import pallas as pl
from jax.experimental.pallas import tpu as pltpu
from jax.experimental.pallas import tpu_sc as plsc

B = 16384
V = 1000001
EMB = 32
SK = 20
OUT_W = 568
NI = OUT_W // 8
NJ = B // 128
NC = 2
NS = 16
L = 16
NW = NC * NS
JPW = NJ // NW
CH = 64
NCHUNK = B // NW // CH


def _assemble(buf, lrows_v, irows_v, emp_v, lat_v, lon_v, sk_v, iota, ones):
    for g in range(CH // L):
        rows = g * L + iota
        for c in range(EMB):
            v = plsc.load_gather(lrows_v, [rows, jnp.full((L,), c, jnp.int32)])
            buf[c // 8, c % 8, pl.ds(g * L, L)] = v
        for c in range(EMB):
            v = plsc.load_gather(irows_v, [rows, jnp.full((L,), c, jnp.int32)])
            buf[4 + c // 8, c % 8, pl.ds(g * L, L)] = v
        buf[8, 0, pl.ds(g * L, L)] = emp_v[pl.ds(g * L, L)]
        buf[8, 1, pl.ds(g * L, L)] = lat_v[pl.ds(g * L, L)]
        buf[8, 2, pl.ds(g * L, L)] = lon_v[pl.ds(g * L, L)]
        for k in range(SK):
            sk = plsc.load_gather(sk_v, [rows, jnp.full((L,), k, jnp.int32)])
            c = 67 + sk
            plsc.store_scatter(buf, [lax.shift_right_logical(c, 3),
                                     lax.bitwise_and(c, 7), rows], ones)


def _clear(buf, sk_v, iota, zeros):
    for g in range(CH // L):
        rows = g * L + iota
        for k in range(SK):
            sk = plsc.load_gather(sk_v, [rows, jnp.full((L,), k, jnp.int32)])
            c = 67 + sk
            plsc.store_scatter(buf, [lax.shift_right_logical(c, 3),
                                     lax.bitwise_and(c, 7), rows], zeros)


def _sc_body(lid_hbm, ind_hbm, emp_hbm, lat_hbm, lon_hbm, sk_hbm,
             ltab_hbm, itab_hbm, out_hbm,
             lid_v, lrows_v, ind_v, emp_v, lat_v, lon_v, sk0_v, sk1_v,
             irows_v, buf0, buf1,
             sem_in, sem_g, sem_o0, sem_o1):
    wid = lax.axis_index("s") * NC + lax.axis_index("c")
    iota = lax.iota(jnp.int32, L)
    ones = jnp.full((L,), 1.0, jnp.float32)
    zeros = jnp.zeros((L,), jnp.float32)

    def zero_i(i, _):
        for k in range(8):
            for g in range(CH // L):
                buf0[i, k, pl.ds(g * L, L)] = zeros
                buf1[i, k, pl.ds(g * L, L)] = zeros
        return 0

    lax.fori_loop(8, NI, zero_i, 0)

    def do_chunk(jj, l0, buf, sk_v, sem_o, first):
        base = jj * 128 + l0
        @pl.when(jnp.logical_not(first))
        def _():
            pltpu.make_async_copy(
                buf, out_hbm.at[:, jj, :, pl.ds(l0, CH)], sem_o).wait()
            _clear(buf, sk_v, iota, zeros)

        cps = [
            pltpu.async_copy(lid_hbm.at[pl.ds(base, CH)], lid_v, sem_in),
            pltpu.async_copy(ind_hbm.at[pl.ds(base, CH)], ind_v, sem_in),
            pltpu.async_copy(emp_hbm.at[pl.ds(base, CH)], emp_v, sem_in),
            pltpu.async_copy(lat_hbm.at[pl.ds(base, CH)], lat_v, sem_in),
            pltpu.async_copy(lon_hbm.at[pl.ds(base, CH)], lon_v, sem_in),
            pltpu.async_copy(sk_hbm.at[pl.ds(base, CH)], sk_v, sem_in),
        ]
        for cp in cps:
            cp.wait()
        g1 = pltpu.async_copy(ltab_hbm.at[lid_v], lrows_v, sem_g)
        g2 = pltpu.async_copy(itab_hbm.at[ind_v], irows_v, sem_g)
        g1.wait()
        g2.wait()
        _assemble(buf, lrows_v, irows_v, emp_v, lat_v, lon_v, sk_v, iota, ones)
        pltpu.async_copy(buf, out_hbm.at[:, jj, :, pl.ds(l0, CH)], sem_o)

    def do_pair(i, _):
        jj = wid * JPW + i
        first = i == 0
        do_chunk(jj, 0, buf0, sk0_v, sem_o0, first)
        do_chunk(jj, CH, buf1, sk1_v, sem_o1, first)
        return 0

    lax.fori_loop(0, JPW, do_pair, 0)
    last_j = wid * JPW + JPW - 1
    pltpu.make_async_copy(
        buf0, out_hbm.at[:, last_j, :, pl.ds(0, CH)], sem_o0).wait()
    pltpu.make_async_copy(
        buf1, out_hbm.at[:, last_j, :, pl.ds(CH, CH)], sem_o1).wait()


@jax.jit
def _run(lid, ind, emp, lat, lon, sk, ltab, itab):
    mesh = plsc.VectorSubcoreMesh(core_axis_name="c", subcore_axis_name="s")
    f = functools.partial(
        pl.kernel,
        mesh=mesh,
        compiler_params=pltpu.CompilerParams(use_tc_tiling_on_sc=False,
                                             needs_layout_passes=False),
        out_type=jax.ShapeDtypeStruct((NI, NJ, 8, 128), jnp.float32),
        scratch_types=[
            pltpu.VMEM((CH,), jnp.int32),
            pltpu.VMEM((CH, 128), jnp.float32),
            pltpu.VMEM((CH,), jnp.int32),
            pltpu.VMEM((CH,), jnp.float32),
            pltpu.VMEM((CH,), jnp.float32),
            pltpu.VMEM((CH,), jnp.float32),
            pltpu.VMEM((CH, SK), jnp.int32),
            pltpu.VMEM((CH, SK), jnp.int32),
            pltpu.VMEM((CH, EMB), jnp.float32),
            pltpu.VMEM((NI, 8, CH), jnp.float32),
            pltpu.VMEM((NI, 8, CH), jnp.float32),
            pltpu.SemaphoreType.DMA,
            pltpu.SemaphoreType.DMA,
            pltpu.SemaphoreType.DMA,
            pltpu.SemaphoreType.DMA,
        ],
    )(_sc_body)
    out4 = f(lid, ind, emp, lat, lon, sk, ltab, itab)
    return jnp.transpose(out4, (1, 3, 0, 2)).reshape(B, OUT_W)


def kernel(listing_id, listing_industry_type, employer_num_employees,
           listing_loc_latitude, listing_loc_longitude, listing_skills,
           listing_table, industry_table):
    tab128 = jnp.pad(listing_table, ((0, 0), (0, 96)))
    return _run(listing_id.astype(jnp.int32),
                listing_industry_type.astype(jnp.int32),
                employer_num_employees,
                listing_loc_latitude,
                listing_loc_longitude,
                listing_skills.astype(jnp.int32),
                tab128, industry_table)


# speedup vs baseline: 13.8655x; 2.9974x over previous
import functools

import jax
import jax.numpy as jnp
from jax import lax
from jax.experimental import pallas as pl
from jax.experimental.pallas import tpu as pltpu
from jax.experimental.pallas import tpu_sc as plsc

B = 16384
V = 1000001
EMB = 32
SK = 20
OUT_W = 568
NI = OUT_W // 8
NJ = B // 128
NC = 2
NS = 16
L = 16
NW = NC * NS
JPW = NJ // NW
CH = 64
NCHUNK = B // NW // CH


def _assemble(buf, irows_v, emp_v, lat_v, lon_v, sk_v, iota, ones):
    for g in range(CH // L):
        rows = g * L + iota
        for c in range(EMB):
            v = plsc.load_gather(irows_v, [rows, jnp.full((L,), c, jnp.int32)])
            buf[4 + c // 8, c % 8, pl.ds(g * L, L)] = v
        buf[8, 0, pl.ds(g * L, L)] = emp_v[pl.ds(g * L, L)]
        buf[8, 1, pl.ds(g * L, L)] = lat_v[pl.ds(g * L, L)]
        buf[8, 2, pl.ds(g * L, L)] = lon_v[pl.ds(g * L, L)]
        for k in range(SK):
            sk = plsc.load_gather(sk_v, [rows, jnp.full((L,), k, jnp.int32)])
            c = 67 + sk
            plsc.store_scatter(buf, [lax.shift_right_logical(c, 3),
                                     lax.bitwise_and(c, 7), rows], ones)


def _clear(buf, sk_v, iota, zeros):
    for g in range(CH // L):
        rows = g * L + iota
        for k in range(SK):
            sk = plsc.load_gather(sk_v, [rows, jnp.full((L,), k, jnp.int32)])
            c = 67 + sk
            plsc.store_scatter(buf, [lax.shift_right_logical(c, 3),
                                     lax.bitwise_and(c, 7), rows], zeros)


def _sc_body(lid_hbm, ind_hbm, emp_hbm, lat_hbm, lon_hbm, sk_hbm,
             ltab_hbm, itab_hbm, out_hbm,
             lid_v, idx_v, ind_v, emp_v, lat_v, lon_v, sk0_v, sk1_v,
             irows_v, buf0, buf1,
             sem_in, sem_g, sem_o0, sem_o1):
    wid = lax.axis_index("s") * NC + lax.axis_index("c")
    iota = lax.iota(jnp.int32, L)
    ones = jnp.full((L,), 1.0, jnp.float32)
    zeros = jnp.zeros((L,), jnp.float32)

    def zero_i(i, _):
        for k in range(8):
            for g in range(CH // L):
                buf0[i, k, pl.ds(g * L, L)] = zeros
                buf1[i, k, pl.ds(g * L, L)] = zeros
        return 0

    lax.fori_loop(8, NI, zero_i, 0)

    def do_chunk(jj, l0, buf, sk_v, sem_o, first):
        base = jj * 128 + l0
        @pl.when(jnp.logical_not(first))
        def _():
            pltpu.make_async_copy(
                buf, out_hbm.at[:, jj, :, pl.ds(l0, CH)], sem_o).wait()
            _clear(buf, sk_v, iota, zeros)

        cps = [
            pltpu.async_copy(lid_hbm.at[pl.ds(base, CH)], lid_v, sem_in),
            pltpu.async_copy(ind_hbm.at[pl.ds(base, CH)], ind_v, sem_in),
            pltpu.async_copy(emp_hbm.at[pl.ds(base, CH)], emp_v, sem_in),
            pltpu.async_copy(lat_hbm.at[pl.ds(base, CH)], lat_v, sem_in),
            pltpu.async_copy(lon_hbm.at[pl.ds(base, CH)], lon_v, sem_in),
            pltpu.async_copy(sk_hbm.at[pl.ds(base, CH)], sk_v, sem_in),
        ]
        for cp in cps:
            cp.wait()
        for g in range(CH // L):
            lid16 = lid_v[pl.ds(g * L, L)]
            b16 = (lax.shift_left(lax.shift_right_logical(lid16, 7), 10)
                   + lax.bitwise_and(lid16, 127))
            for c in range(EMB):
                off = (c // 8) * 8000512 + (c % 8) * 128
                idx_v[c, pl.ds(g * L, L)] = b16 + jnp.int32(off)
        gs = [pltpu.async_copy(ltab_hbm.at[idx_v.at[c]],
                               buf.at[c // 8, c % 8], sem_g)
              for c in range(EMB)]
        g2 = pltpu.async_copy(itab_hbm.at[ind_v], irows_v, sem_g)
        for cp in gs:
            cp.wait()
        g2.wait()
        _assemble(buf, irows_v, emp_v, lat_v, lon_v, sk_v, iota, ones)
        pltpu.async_copy(buf, out_hbm.at[:, jj, :, pl.ds(l0, CH)], sem_o)

    def do_pair(i, _):
        jj = wid * JPW + i
        first = i == 0
        do_chunk(jj, 0, buf0, sk0_v, sem_o0, first)
        do_chunk(jj, CH, buf1, sk1_v, sem_o1, first)
        return 0

    lax.fori_loop(0, JPW, do_pair, 0)
    last_j = wid * JPW + JPW - 1
    pltpu.make_async_copy(
        buf0, out_hbm.at[:, last_j, :, pl.ds(0, CH)], sem_o0).wait()
    pltpu.make_async_copy(
        buf1, out_hbm.at[:, last_j, :, pl.ds(CH, CH)], sem_o1).wait()


@jax.jit
def _run(lid, ind, emp, lat, lon, sk, ltab, itab):
    mesh = plsc.VectorSubcoreMesh(core_axis_name="c", subcore_axis_name="s")
    f = functools.partial(
        pl.kernel,
        mesh=mesh,
        compiler_params=pltpu.CompilerParams(use_tc_tiling_on_sc=False,
                                             needs_layout_passes=False),
        out_type=jax.ShapeDtypeStruct((NI, NJ, 8, 128), jnp.float32),
        scratch_types=[
            pltpu.VMEM((CH,), jnp.int32),
            pltpu.VMEM((EMB, CH), jnp.int32),
            pltpu.VMEM((CH,), jnp.int32),
            pltpu.VMEM((CH,), jnp.float32),
            pltpu.VMEM((CH,), jnp.float32),
            pltpu.VMEM((CH,), jnp.float32),
            pltpu.VMEM((CH, SK), jnp.int32),
            pltpu.VMEM((CH, SK), jnp.int32),
            pltpu.VMEM((CH, EMB), jnp.float32),
            pltpu.VMEM((NI, 8, CH), jnp.float32),
            pltpu.VMEM((NI, 8, CH), jnp.float32),
            pltpu.SemaphoreType.DMA,
            pltpu.SemaphoreType.DMA,
            pltpu.SemaphoreType.DMA,
            pltpu.SemaphoreType.DMA,
        ],
    )(_sc_body)
    out4 = f(lid, ind, emp, lat, lon, sk, ltab, itab)
    return jnp.transpose(out4, (1, 3, 0, 2)).reshape(B, OUT_W)


def kernel(listing_id, listing_industry_type, employer_num_employees,
           listing_loc_latitude, listing_loc_longitude, listing_skills,
           listing_table, industry_table):
    padded = jnp.pad(listing_table, ((0, 63), (0, 0)))
    tab_img = padded.T.reshape(4, 8, 7813, 128).transpose(0, 2, 1, 3).reshape(-1)
    return _run(listing_id.astype(jnp.int32),
                listing_industry_type.astype(jnp.int32),
                employer_num_employees,
                listing_loc_latitude,
                listing_loc_longitude,
                listing_skills.astype(jnp.int32),
                tab_img, industry_table)


# speedup vs baseline: 14.7009x; 1.0603x over previous
import functools

import jax
import jax.numpy as jnp
from jax import lax
from jax.experimental import pallas as pl
from jax.experimental.pallas import tpu as pltpu
from jax.experimental.pallas import tpu_sc as plsc

B = 16384
V = 1000001
EMB = 32
SK = 20
OUT_W = 568
NI = OUT_W // 8
NJ = B // 128
NC = 2
NS = 16
L = 16
NW = NC * NS
JPW = NJ // NW
CH = 64
NCHUNK = B // NW // CH


def _assemble(buf, irows_v, emp_v, lat_v, lon_v, sk_v, iota, ones):
    for g in range(CH // L):
        rows = g * L + iota
        for c in range(EMB):
            v = plsc.load_gather(irows_v, [rows, jnp.full((L,), c, jnp.int32)])
            buf[4 + c // 8, c % 8, pl.ds(g * L, L)] = v
        buf[8, 0, pl.ds(g * L, L)] = emp_v[pl.ds(g * L, L)]
        buf[8, 1, pl.ds(g * L, L)] = lat_v[pl.ds(g * L, L)]
        buf[8, 2, pl.ds(g * L, L)] = lon_v[pl.ds(g * L, L)]
        for k in range(SK):
            sk = plsc.load_gather(sk_v, [rows, jnp.full((L,), k, jnp.int32)])
            c = 67 + sk
            plsc.store_scatter(buf, [lax.shift_right_logical(c, 3),
                                     lax.bitwise_and(c, 7), rows], ones)


def _clear(buf, sk_v, iota, zeros):
    for g in range(CH // L):
        rows = g * L + iota
        for k in range(SK):
            sk = plsc.load_gather(sk_v, [rows, jnp.full((L,), k, jnp.int32)])
            c = 67 + sk
            plsc.store_scatter(buf, [lax.shift_right_logical(c, 3),
                                     lax.bitwise_and(c, 7), rows], zeros)


def _g_body(lid_hbm, ltab_hbm, out_hbm,
            lid_v, idx_v, gbuf0, gbuf1,
            sem_in, sem_g, sem_o0, sem_o1):
    wid = lax.axis_index("s") * NC + lax.axis_index("c")

    def do_chunk(jj, l0, gbuf, sem_o, first):
        base = jj * 128 + l0

        @pl.when(jnp.logical_not(first))
        def _():
            pltpu.make_async_copy(
                gbuf, out_hbm.at[pl.ds(0, 4), jj, :, pl.ds(l0, CH)],
                sem_o).wait()

        pltpu.async_copy(lid_hbm.at[pl.ds(base, CH)], lid_v, sem_in).wait()
        for g in range(CH // L):
            lid16 = lid_v[pl.ds(g * L, L)]
            b16 = (lax.shift_left(lax.shift_right_logical(lid16, 7), 10)
                   + lax.bitwise_and(lid16, 127))
            for c in range(EMB):
                off = (c // 8) * 8000512 + (c % 8) * 128
                idx_v[c, pl.ds(g * L, L)] = b16 + jnp.int32(off)
        gs = [pltpu.async_copy(ltab_hbm.at[idx_v.at[c]],
                               gbuf.at[c // 8, c % 8], sem_g)
              for c in range(EMB)]
        for cp in gs:
            cp.wait()
        pltpu.async_copy(gbuf, out_hbm.at[pl.ds(0, 4), jj, :, pl.ds(l0, CH)],
                         sem_o)

    def do_pair(i, _):
        jj = wid * JPW + i
        first = i == 0
        do_chunk(jj, 0, gbuf0, sem_o0, first)
        do_chunk(jj, CH, gbuf1, sem_o1, first)
        return 0

    lax.fori_loop(0, JPW, do_pair, 0)
    last_j = wid * JPW + JPW - 1
    pltpu.make_async_copy(
        gbuf0, out_hbm.at[pl.ds(0, 4), last_j, :, pl.ds(0, CH)], sem_o0).wait()
    pltpu.make_async_copy(
        gbuf1, out_hbm.at[pl.ds(0, 4), last_j, :, pl.ds(CH, CH)], sem_o1).wait()


def _sc_body(ind_hbm, emp_hbm, lat_hbm, lon_hbm, sk_hbm,
             itab_hbm, out_hbm,
             ind_v, emp_v, lat_v, lon_v, sk0_v, sk1_v,
             irows_v, buf0, buf1,
             sem_in, sem_g, sem_o0, sem_o1):
    wid = lax.axis_index("s") * NC + lax.axis_index("c")
    iota = lax.iota(jnp.int32, L)
    ones = jnp.full((L,), 1.0, jnp.float32)
    zeros = jnp.zeros((L,), jnp.float32)

    def zero_i(i, _):
        for k in range(8):
            for g in range(CH // L):
                buf0[i, k, pl.ds(g * L, L)] = zeros
                buf1[i, k, pl.ds(g * L, L)] = zeros
        return 0

    lax.fori_loop(8, NI, zero_i, 0)

    def do_chunk(jj, l0, buf, sk_v, sem_o, first):
        base = jj * 128 + l0
        @pl.when(jnp.logical_not(first))
        def _():
            pltpu.make_async_copy(
                buf, out_hbm.at[:, jj, :, pl.ds(l0, CH)], sem_o).wait()
            _clear(buf, sk_v, iota, zeros)

        cps = [
            pltpu.async_copy(ind_hbm.at[pl.ds(base, CH)], ind_v, sem_in),
            pltpu.async_copy(emp_hbm.at[pl.ds(base, CH)], emp_v, sem_in),
            pltpu.async_copy(lat_hbm.at[pl.ds(base, CH)], lat_v, sem_in),
            pltpu.async_copy(lon_hbm.at[pl.ds(base, CH)], lon_v, sem_in),
            pltpu.async_copy(sk_hbm.at[pl.ds(base, CH)], sk_v, sem_in),
        ]
        for cp in cps:
            cp.wait()
        g2 = pltpu.async_copy(itab_hbm.at[ind_v], irows_v, sem_g)
        g2.wait()
        _assemble(buf, irows_v, emp_v, lat_v, lon_v, sk_v, iota, ones)
        pltpu.async_copy(buf, out_hbm.at[:, jj, :, pl.ds(l0, CH)], sem_o)

    def do_pair(i, _):
        jj = wid * JPW + i
        first = i == 0
        do_chunk(jj, 0, buf0, sk0_v, sem_o0, first)
        do_chunk(jj, CH, buf1, sk1_v, sem_o1, first)
        return 0

    lax.fori_loop(0, JPW, do_pair, 0)
    last_j = wid * JPW + JPW - 1
    pltpu.make_async_copy(
        buf0, out_hbm.at[:, last_j, :, pl.ds(0, CH)], sem_o0).wait()
    pltpu.make_async_copy(
        buf1, out_hbm.at[:, last_j, :, pl.ds(CH, CH)], sem_o1).wait()


@jax.jit
def _run(lid, ind, emp, lat, lon, sk, ltab, itab):
    mesh = plsc.VectorSubcoreMesh(core_axis_name="c", subcore_axis_name="s")
    f = functools.partial(
        pl.kernel,
        mesh=mesh,
        compiler_params=pltpu.CompilerParams(use_tc_tiling_on_sc=False,
                                             needs_layout_passes=False),
        out_type=jax.ShapeDtypeStruct((NI, NJ, 8, 128), jnp.float32),
        scratch_types=[
            pltpu.VMEM((CH,), jnp.int32),
            pltpu.VMEM((CH,), jnp.float32),
            pltpu.VMEM((CH,), jnp.float32),
            pltpu.VMEM((CH,), jnp.float32),
            pltpu.VMEM((CH, SK), jnp.int32),
            pltpu.VMEM((CH, SK), jnp.int32),
            pltpu.VMEM((CH, EMB), jnp.float32),
            pltpu.VMEM((NI, 8, CH), jnp.float32),
            pltpu.VMEM((NI, 8, CH), jnp.float32),
            pltpu.SemaphoreType.DMA,
            pltpu.SemaphoreType.DMA,
            pltpu.SemaphoreType.DMA,
            pltpu.SemaphoreType.DMA,
        ],
    )(_sc_body)
    out4 = f(ind, emp, lat, lon, sk, itab)

    gk = functools.partial(
        pl.kernel,
        mesh=mesh,
        compiler_params=pltpu.CompilerParams(use_tc_tiling_on_sc=False,
                                             needs_layout_passes=False),
        out_type=(),
        scratch_types=[
            pltpu.VMEM((CH,), jnp.int32),
            pltpu.VMEM((EMB, CH), jnp.int32),
            pltpu.VMEM((4, 8, CH), jnp.float32),
            pltpu.VMEM((4, 8, CH), jnp.float32),
            pltpu.SemaphoreType.DMA,
            pltpu.SemaphoreType.DMA,
            pltpu.SemaphoreType.DMA,
            pltpu.SemaphoreType.DMA,
        ],
    )(_g_body)
    out_ref = jax.new_ref(out4)
    gk(lid, ltab, out_ref)
    out4b = out_ref[...]
    return jnp.transpose(out4b, (1, 3, 0, 2)).reshape(B, OUT_W)


def kernel(listing_id, listing_industry_type, employer_num_employees,
           listing_loc_latitude, listing_loc_longitude, listing_skills,
           listing_table, industry_table):
    padded = jnp.pad(listing_table, ((0, 63), (0, 0)))
    tab_img = padded.T.reshape(4, 8, 7813, 128).transpose(0, 2, 1, 3).reshape(-1)
    return _run(listing_id.astype(jnp.int32),
                listing_industry_type.astype(jnp.int32),
                employer_num_employees,
                listing_loc_latitude,
                listing_loc_longitude,
                listing_skills.astype(jnp.int32),
                tab_img, industry_table)


# speedup vs baseline: 16.9423x; 1.1525x over previous
import functools

import jax
import jax.numpy as jnp
from jax import lax
from jax.experimental import pallas as pl
from jax.experimental.pallas import tpu as pltpu
from jax.experimental.pallas import tpu_sc as plsc

B = 16384
EMB = 32
SK = 20
OUT_W = 568
NI = OUT_W // 8
NJ = B // 128
NC = 2
NS = 16
L = 16
NW = NC * NS
JPW = NJ // NW
CH = 64
TBJ = 7813
SBJ = 128


def _g_body(lid_hbm, ltab_hbm, out_hbm,
            lid_v, idx_v, gbuf0, gbuf1,
            sem_in, sem_g, sem_o0, sem_o1):
    wid = lax.axis_index("s") * NC + lax.axis_index("c")

    def do_j(jj, gbuf, sem_o, first):
        @pl.when(jnp.logical_not(first))
        def _():
            pltpu.make_async_copy(
                gbuf, out_hbm.at[pl.ds(0, 4), jj, :, :], sem_o).wait()

        pltpu.async_copy(lid_hbm.at[pl.ds(jj * 128, 128)], lid_v,
                         sem_in).wait()
        for g in range(128 // L):
            lid16 = lid_v[pl.ds(g * L, L)]
            b16 = (lax.shift_left(lax.shift_right_logical(lid16, 7), 10)
                   + lax.bitwise_and(lid16, 127))
            for c in range(EMB):
                off = (c // 8) * (TBJ * 1024) + (c % 8) * 128
                idx_v[c, pl.ds(g * L, L)] = b16 + jnp.int32(off)
        gs = [pltpu.async_copy(ltab_hbm.at[idx_v.at[c]],
                               gbuf.at[c // 8, c % 8], sem_g)
              for c in range(EMB)]
        for cp in gs:
            cp.wait()
        pltpu.async_copy(gbuf, out_hbm.at[pl.ds(0, 4), jj, :, :], sem_o)

    def do_pair(t, _):
        jj = wid * JPW + 2 * t
        first = t == 0
        do_j(jj, gbuf0, sem_o0, first)
        do_j(jj + 1, gbuf1, sem_o1, first)
        return 0

    lax.fori_loop(0, JPW // 2, do_pair, 0)
    last_j = wid * JPW + JPW - 2
    pltpu.make_async_copy(
        gbuf0, out_hbm.at[pl.ds(0, 4), last_j, :, :], sem_o0).wait()
    pltpu.make_async_copy(
        gbuf1, out_hbm.at[pl.ds(0, 4), last_j + 1, :, :], sem_o1).wait()


def _multihot(buf, sk_v, l0, iota, val):
    for g in range(CH // L):
        rows = g * L + iota
        for k in range(SK):
            sk = sk_v[k // 8, pl.ds((k % 8) * 128 + l0 + g * L, L)]
            c = 67 + sk
            plsc.store_scatter(buf, [lax.shift_right_logical(c, 3),
                                     lax.bitwise_and(c, 7), rows], val)


def _assemble(buf, irows_v, emp_v, lat_v, lon_v, sk_v, l0, iota, ones):
    for g in range(CH // L):
        rows = g * L + iota
        for c in range(EMB):
            v = plsc.load_gather(irows_v,
                                 [l0 + rows, jnp.full((L,), c, jnp.int32)])
            buf[4 + c // 8, c % 8, pl.ds(g * L, L)] = v
        buf[8, 0, pl.ds(g * L, L)] = emp_v[pl.ds(l0 + g * L, L)]
        buf[8, 1, pl.ds(g * L, L)] = lat_v[pl.ds(l0 + g * L, L)]
        buf[8, 2, pl.ds(g * L, L)] = lon_v[pl.ds(l0 + g * L, L)]
    _multihot(buf, sk_v, l0, iota, ones)


def _sc_body(ind_hbm, emp_hbm, lat_hbm, lon_hbm, sk_hbm,
             itab_hbm, out_hbm,
             ind_v, emp_v, lat_v, lon_v, sk_v,
             irows_v, buf0, buf1,
             sem_in, sem_g, sem_o0, sem_o1):
    wid = lax.axis_index("s") * NC + lax.axis_index("c")
    iota = lax.iota(jnp.int32, L)
    ones = jnp.full((L,), 1.0, jnp.float32)
    zeros = jnp.zeros((L,), jnp.float32)

    def zero_i(i, _):
        for k in range(8):
            for g in range(CH // L):
                buf0[i, k, pl.ds(g * L, L)] = zeros
                buf1[i, k, pl.ds(g * L, L)] = zeros
        return 0

    lax.fori_loop(8, NI, zero_i, 0)

    def do_pair(t, _):
        jj = wid * JPW + t
        first = t == 0

        @pl.when(jnp.logical_not(first))
        def _():
            pltpu.make_async_copy(
                buf0, out_hbm.at[:, jj, :, pl.ds(0, CH)],
                sem_o0).wait()
            pltpu.make_async_copy(
                buf1, out_hbm.at[:, jj, :, pl.ds(CH, CH)],
                sem_o1).wait()
            _multihot(buf0, sk_v, 0, iota, zeros)
            _multihot(buf1, sk_v, CH, iota, zeros)

        base = jj * 128
        cps = [
            pltpu.async_copy(ind_hbm.at[pl.ds(base, 128)], ind_v, sem_in),
            pltpu.async_copy(emp_hbm.at[pl.ds(base, 128)], emp_v, sem_in),
            pltpu.async_copy(lat_hbm.at[pl.ds(base, 128)], lat_v, sem_in),
            pltpu.async_copy(lon_hbm.at[pl.ds(base, 128)], lon_v, sem_in),
        ]
        cps += [
            pltpu.async_copy(
                sk_hbm.at[pl.ds(i * (SBJ * 1024) + jj * 1024, 1024)],
                sk_v.at[i], sem_in)
            for i in range(3)
        ]
        for cp in cps:
            cp.wait()
        g2 = pltpu.async_copy(itab_hbm.at[ind_v], irows_v, sem_g)
        g2.wait()
        _assemble(buf0, irows_v, emp_v, lat_v, lon_v, sk_v, 0, iota, ones)
        pltpu.async_copy(
            buf0, out_hbm.at[:, jj, :, pl.ds(0, CH)], sem_o0)
        _assemble(buf1, irows_v, emp_v, lat_v, lon_v, sk_v, CH, iota, ones)
        pltpu.async_copy(
            buf1, out_hbm.at[:, jj, :, pl.ds(CH, CH)], sem_o1)
        return 0

    lax.fori_loop(0, JPW, do_pair, 0)
    last_j = wid * JPW + JPW - 1
    pltpu.make_async_copy(
        buf0, out_hbm.at[:, last_j, :, pl.ds(0, CH)],
        sem_o0).wait()
    pltpu.make_async_copy(
        buf1, out_hbm.at[:, last_j, :, pl.ds(CH, CH)],
        sem_o1).wait()


@jax.jit
def _run(lid, ind, emp, lat, lon, sk_img, ltab_img, itab):
    mesh = plsc.VectorSubcoreMesh(core_axis_name="c", subcore_axis_name="s")
    f = functools.partial(
        pl.kernel,
        mesh=mesh,
        compiler_params=pltpu.CompilerParams(use_tc_tiling_on_sc=False,
                                             needs_layout_passes=False),
        out_type=jax.ShapeDtypeStruct((NI, NJ, 8, 128), jnp.float32),
        scratch_types=[
            pltpu.VMEM((128,), jnp.int32),
            pltpu.VMEM((128,), jnp.float32),
            pltpu.VMEM((128,), jnp.float32),
            pltpu.VMEM((128,), jnp.float32),
            pltpu.VMEM((3, 1024), jnp.int32),
            pltpu.VMEM((128, EMB), jnp.float32),
            pltpu.VMEM((NI, 8, CH), jnp.float32),
            pltpu.VMEM((NI, 8, CH), jnp.float32),
            pltpu.SemaphoreType.DMA,
            pltpu.SemaphoreType.DMA,
            pltpu.SemaphoreType.DMA,
            pltpu.SemaphoreType.DMA,
        ],
    )(_sc_body)
    out4 = f(ind, emp, lat, lon, sk_img, itab)

    gk = functools.partial(
        pl.kernel,
        mesh=mesh,
        compiler_params=pltpu.CompilerParams(use_tc_tiling_on_sc=False,
                                             needs_layout_passes=False),
        out_type=(),
        scratch_types=[
            pltpu.VMEM((128,), jnp.int32),
            pltpu.VMEM((EMB, 128), jnp.int32),
            pltpu.VMEM((4, 8, 128), jnp.float32),
            pltpu.VMEM((4, 8, 128), jnp.float32),
            pltpu.SemaphoreType.DMA,
            pltpu.SemaphoreType.DMA,
            pltpu.SemaphoreType.DMA,
            pltpu.SemaphoreType.DMA,
        ],
    )(_g_body)
    out_ref = jax.new_ref(out4)
    gk(lid, ltab_img, out_ref)
    out4b = out_ref[...]
    return jnp.transpose(out4b, (1, 3, 0, 2)).reshape(B, OUT_W)


def _tiled_image(x, pad_rows):
    n, m = x.shape
    padded = jnp.pad(x, ((0, pad_rows), (0, 0)))
    bi = m // 8
    bj = (n + pad_rows) // 128
    return (padded.T.reshape(bi, 8, bj, 128)
            .transpose(0, 2, 1, 3).reshape(-1))


def kernel(listing_id, listing_industry_type, employer_num_employees,
           listing_loc_latitude, listing_loc_longitude, listing_skills,
           listing_table, industry_table):
    tab_img = _tiled_image(listing_table, 63)
    sk_img = _tiled_image(
        jnp.pad(listing_skills.astype(jnp.int32), ((0, 0), (0, 4))), 0)
    return _run(listing_id.astype(jnp.int32),
                listing_industry_type.astype(jnp.int32),
                employer_num_employees,
                listing_loc_latitude,
                listing_loc_longitude,
                sk_img, tab_img, industry_table)
